# baseline (speedup 1.0000x reference)
import functools

import jax
import jax.numpy as jnp
from jax import lax
from jax.experimental import pallas as pl
from jax.experimental.pallas import tpu as pltpu
from jax.experimental.pallas import tpu_sc as plsc

_NUM_CORES = 2
_NUM_SUBCORES = 16
_NUM_WORKERS = _NUM_CORES * _NUM_SUBCORES
_CHUNK = 64


def kernel(x, table):
    (batch,) = x.shape
    _, dim = table.shape
    b_per_w = batch // _NUM_WORKERS
    n_chunks = b_per_w // _CHUNK

    mesh = plsc.VectorSubcoreMesh(core_axis_name="c", subcore_axis_name="s")

    @functools.partial(
        pl.kernel,
        mesh=mesh,
        out_type=jax.ShapeDtypeStruct((batch, dim), table.dtype),
        scratch_types=[
            pltpu.VMEM((b_per_w,), jnp.int32),
            pltpu.VMEM((_CHUNK, dim), table.dtype),
            pltpu.VMEM((_CHUNK, dim), table.dtype),
            pltpu.SemaphoreType.DMA,
            pltpu.SemaphoreType.DMA,
            pltpu.SemaphoreType.DMA,
            pltpu.SemaphoreType.DMA,
        ],
    )
    def gather_kernel(table_hbm, idx_hbm, out_hbm, idx_v, buf0, buf1,
                      g0, g1, w0, w1):
        wid = lax.axis_index("s") * _NUM_CORES + lax.axis_index("c")
        base = wid * b_per_w
        pltpu.sync_copy(idx_hbm.at[pl.ds(base, b_per_w)], idx_v)

        bufs = (buf0, buf1)
        gsems = (g0, g1)
        wsems = (w0, w1)
        gathers = [None] * n_chunks
        writes = [None] * n_chunks
        for i in range(n_chunks):
            b = i % 2
            if i >= 2:
                writes[i - 2].wait()
            gathers[i] = pltpu.async_copy(
                table_hbm.at[idx_v.at[pl.ds(i * _CHUNK, _CHUNK)]],
                bufs[b], gsems[b])
            if i >= 1:
                gathers[i - 1].wait()
                writes[i - 1] = pltpu.async_copy(
                    bufs[(i - 1) % 2],
                    out_hbm.at[pl.ds(base + (i - 1) * _CHUNK, _CHUNK)],
                    wsems[(i - 1) % 2])
        last = n_chunks - 1
        gathers[last].wait()
        writes[last] = pltpu.async_copy(
            bufs[last % 2],
            out_hbm.at[pl.ds(base + last * _CHUNK, _CHUNK)],
            wsems[last % 2])
        if n_chunks >= 2:
            writes[last - 1].wait()
        writes[last].wait()

    return gather_kernel(table, x)


# speedup vs baseline: 1.0731x; 1.0731x over previous
import functools

import jax
import jax.numpy as jnp
from jax import lax
from jax.experimental import pallas as pl
from jax.experimental.pallas import tpu as pltpu
from jax.experimental.pallas import tpu_sc as plsc

_NUM_CORES = 2
_NUM_SUBCORES = 16
_NUM_WORKERS = _NUM_CORES * _NUM_SUBCORES
_CHUNK = 256


def kernel(x, table):
    (batch,) = x.shape
    _, dim = table.shape
    b_per_w = batch // _NUM_WORKERS
    n_chunks = b_per_w // _CHUNK

    mesh = plsc.VectorSubcoreMesh(core_axis_name="c", subcore_axis_name="s")

    @functools.partial(
        pl.kernel,
        mesh=mesh,
        out_type=jax.ShapeDtypeStruct((batch, dim), table.dtype),
        scratch_types=[
            pltpu.VMEM((b_per_w,), jnp.int32),
            pltpu.VMEM((_CHUNK, dim), table.dtype),
            pltpu.VMEM((_CHUNK, dim), table.dtype),
            pltpu.SemaphoreType.DMA,
            pltpu.SemaphoreType.DMA,
            pltpu.SemaphoreType.DMA,
            pltpu.SemaphoreType.DMA,
        ],
    )
    def gather_kernel(table_hbm, idx_hbm, out_hbm, idx_v, buf0, buf1,
                      g0, g1, w0, w1):
        wid = lax.axis_index("s") * _NUM_CORES + lax.axis_index("c")
        base = wid * b_per_w
        pltpu.sync_copy(idx_hbm.at[pl.ds(base, b_per_w)], idx_v)

        bufs = (buf0, buf1)
        gsems = (g0, g1)
        wsems = (w0, w1)
        gathers = [None] * n_chunks
        writes = [None] * n_chunks
        for i in range(n_chunks):
            b = i % 2
            if i >= 2:
                writes[i - 2].wait()
            gathers[i] = pltpu.async_copy(
                table_hbm.at[idx_v.at[pl.ds(i * _CHUNK, _CHUNK)]],
                bufs[b], gsems[b])
            if i >= 1:
                gathers[i - 1].wait()
                writes[i - 1] = pltpu.async_copy(
                    bufs[(i - 1) % 2],
                    out_hbm.at[pl.ds(base + (i - 1) * _CHUNK, _CHUNK)],
                    wsems[(i - 1) % 2])
        last = n_chunks - 1
        gathers[last].wait()
        writes[last] = pltpu.async_copy(
            bufs[last % 2],
            out_hbm.at[pl.ds(base + last * _CHUNK, _CHUNK)],
            wsems[last % 2])
        if n_chunks >= 2:
            writes[last - 1].wait()
        writes[last].wait()

    return gather_kernel(table, x)
